# baseline (speedup 1.0000x reference)
import jax
import jax.numpy as jnp
from jax import lax
from jax.experimental import pallas as pl
from jax.experimental.pallas import tpu as pltpu
from jax.experimental.pallas import tpu_sc as plsc

NN = 10000
NE = 320000
DIN = 128
H = 64
W128 = 128
NC = 2
NS = 16
NW = NC * NS
CHE = 128
NCHUNK = NE // CHE
_F32 = jnp.float32

_MESH = plsc.VectorSubcoreMesh(core_axis_name="c", subcore_axis_name="s")

_RPW = 624
_TAIL = NN - NS * _RPW


NFULL = NCHUNK // NW
NEXTRA = NCHUNK - NFULL * NW


def _worker_ids():
    c = lax.axis_index("c")
    s = lax.axis_index("s")
    return c, s, s * NC + c


def _zero_acc(z2d, acc, s):
    pltpu.sync_copy(z2d.at[pl.ds(s * _RPW, _RPW)],
                    acc.at[pl.ds(s * _RPW, _RPW)])

    @pl.when(s == 0)
    def _():
        pltpu.sync_copy(z2d.at[pl.ds(NS * _RPW, _TAIL)],
                        acc.at[pl.ds(NS * _RPW, _TAIL)])


KPW = 80
PH = 40


def _sc_agg_body(tab, src3, dst3, z2d,
                 agg_out,
                 idxs, idxd, rows_a, rows_b,
                 sem_a, sem_b, sem_sa, sem_sb, acc):
    c, s, w = _worker_ids()
    _zero_acc(z2d, acc, s)
    plsc.subcore_barrier()

    for phase in range(KPW // PH):
        pltpu.sync_copy(src3.at[w, pl.ds(phase * PH, PH)], idxs)
        pltpu.sync_copy(dst3.at[w, pl.ds(phase * PH, PH)], idxd)

        def body(k, carry):
            pltpu.async_copy(tab.at[idxs.at[k]], rows_a, sem_a).wait()
            pltpu.sync_copy(rows_a, acc.at[idxd.at[k]], add=True)
            return carry

        lax.fori_loop(0, PH, body, 0)

    plsc.subcore_barrier()

    pltpu.sync_copy(acc.at[pl.ds(s * _RPW, _RPW)],
                    agg_out.at[c, pl.ds(s * _RPW, _RPW)])

    @pl.when(s == 0)
    def _():
        pltpu.sync_copy(acc.at[pl.ds(NS * _RPW, _TAIL)],
                        agg_out.at[c, pl.ds(NS * _RPW, _TAIL)])


def _sc_gather_body(tab, src3, dst3,
                    gs_out, gd_out,
                    idxs_all, idxd_all,
                    rows_sa, rows_sb, rows_da, rows_db,
                    sem_sa, sem_sb, sem_da, sem_db):
    c, s, w = _worker_ids()
    pltpu.sync_copy(src3.at[w], idxs_all)
    pltpu.sync_copy(dst3.at[w], idxd_all)

    pltpu.async_copy(tab.at[idxs_all.at[0]], rows_sa, sem_sa)
    pltpu.async_copy(tab.at[idxd_all.at[0]], rows_da, sem_da)
    pltpu.async_copy(tab.at[idxs_all.at[1]], rows_sb, sem_sb)
    pltpu.async_copy(tab.at[idxd_all.at[1]], rows_db, sem_db)

    def chunk_of(k):
        return k * NW + w

    def body(m, carry):
        kA = 2 * m
        kB = kA + 1
        baseA = chunk_of(kA) * CHE
        baseB = chunk_of(kB) * CHE
        pltpu.make_async_copy(tab.at[idxs_all.at[kA]], rows_sa, sem_sa).wait()
        pltpu.sync_copy(rows_sa, gs_out.at[pl.ds(baseA, CHE)])
        pltpu.make_async_copy(tab.at[idxd_all.at[kA]], rows_da, sem_da).wait()
        pltpu.sync_copy(rows_da, gd_out.at[pl.ds(baseA, CHE)])

        @pl.when(m < NFULL // 2 - 1)
        def _():
            pltpu.async_copy(tab.at[idxs_all.at[kA + 2]], rows_sa, sem_sa)
            pltpu.async_copy(tab.at[idxd_all.at[kA + 2]], rows_da, sem_da)

        pltpu.make_async_copy(tab.at[idxs_all.at[kB]], rows_sb, sem_sb).wait()
        pltpu.sync_copy(rows_sb, gs_out.at[pl.ds(baseB, CHE)])
        pltpu.make_async_copy(tab.at[idxd_all.at[kB]], rows_db, sem_db).wait()
        pltpu.sync_copy(rows_db, gd_out.at[pl.ds(baseB, CHE)])

        @pl.when(m < NFULL // 2 - 1)
        def _():
            pltpu.async_copy(tab.at[idxs_all.at[kB + 2]], rows_sb, sem_sb)
            pltpu.async_copy(tab.at[idxd_all.at[kB + 2]], rows_db, sem_db)

        return carry

    lax.fori_loop(0, NFULL // 2, body, 0)

    @pl.when(w < NEXTRA)
    def _():
        base = (NFULL * NW + w) * CHE
        pltpu.async_copy(tab.at[idxs_all.at[NFULL]], rows_sa, sem_sa).wait()
        pltpu.sync_copy(rows_sa, gs_out.at[pl.ds(base, CHE)])
        pltpu.async_copy(tab.at[idxd_all.at[NFULL]], rows_da, sem_da).wait()
        pltpu.sync_copy(rows_da, gd_out.at[pl.ds(base, CHE)])


_sc_agg = pl.kernel(
    _sc_agg_body,
    out_type=jax.ShapeDtypeStruct((NC, NN, W128), _F32),
    mesh=_MESH,
    scratch_types=[
        pltpu.VMEM((PH, CHE), jnp.int32),
        pltpu.VMEM((PH, CHE), jnp.int32),
        pltpu.VMEM((CHE, W128), _F32),
        pltpu.VMEM((CHE, W128), _F32),
        pltpu.SemaphoreType.DMA,
        pltpu.SemaphoreType.DMA,
        pltpu.SemaphoreType.DMA,
        pltpu.SemaphoreType.DMA,
        pltpu.VMEM_SHARED((NN + CHE, W128), _F32),
    ],
)

_sc_gather = pl.kernel(
    _sc_gather_body,
    out_type=(jax.ShapeDtypeStruct((NE, W128), _F32),
              jax.ShapeDtypeStruct((NE, W128), _F32)),
    mesh=_MESH,
    scratch_types=[
        pltpu.VMEM((KPW, CHE), jnp.int32),
        pltpu.VMEM((KPW, CHE), jnp.int32),
        pltpu.VMEM((CHE, W128), _F32),
        pltpu.VMEM((CHE, W128), _F32),
        pltpu.VMEM((CHE, W128), _F32),
        pltpu.VMEM((CHE, W128), _F32),
        pltpu.SemaphoreType.DMA,
        pltpu.SemaphoreType.DMA,
        pltpu.SemaphoreType.DMA,
        pltpu.SemaphoreType.DMA,
    ],
)


def _mm0_body(x_ref, wpad_ref, e64_ref, wr_ref, br_ref, tab_ref, sf_ref):
    x = x_ref[...]
    tab_ref[...] = jnp.dot(x, wpad_ref[...],
                           preferred_element_type=_F32) + e64_ref[...]
    sf_ref[...] = jnp.dot(x, wr_ref[...],
                          preferred_element_type=_F32) + br_ref[...]


_mm0 = pl.pallas_call(
    _mm0_body,
    out_shape=(jax.ShapeDtypeStruct((NN, W128), _F32),
               jax.ShapeDtypeStruct((NN, H), _F32)),
)


def _combine_first_body(p_ref, sf_ref, wpad_ref, wr_ref, br_ref,
                        tab_ref, sfo_ref, inv_ref):
    agg = p_ref[0] + p_ref[1]
    inv = 1.0 / jnp.maximum(agg[:, H:H + 1], 1.0)
    h = jnp.maximum(agg[:, :H] * inv + sf_ref[...], 0.0)
    tab_ref[...] = jnp.dot(h, wpad_ref[...], preferred_element_type=_F32)
    sfo_ref[...] = jnp.dot(h, wr_ref[...],
                           preferred_element_type=_F32) + br_ref[...]
    inv_ref[...] = inv


_combine_first = pl.pallas_call(
    _combine_first_body,
    out_shape=(jax.ShapeDtypeStruct((NN, W128), _F32),
               jax.ShapeDtypeStruct((NN, H), _F32),
               jax.ShapeDtypeStruct((NN, 1), _F32)),
)


def _combine_mid_body(p_ref, inv_ref, sf_ref, wpad_ref, wr_ref, br_ref,
                      tab_ref, sfo_ref):
    agg = p_ref[0] + p_ref[1]
    h = jnp.maximum(agg[:, :H] * inv_ref[...] + sf_ref[...], 0.0)
    tab_ref[...] = jnp.dot(h, wpad_ref[...], preferred_element_type=_F32)
    sfo_ref[...] = jnp.dot(h, wr_ref[...],
                           preferred_element_type=_F32) + br_ref[...]


_combine_mid = pl.pallas_call(
    _combine_mid_body,
    out_shape=(jax.ShapeDtypeStruct((NN, W128), _F32),
               jax.ShapeDtypeStruct((NN, H), _F32)),
)


def _combine_last_body(p_ref, inv_ref, sf_ref, wc_ref, tab_ref):
    agg = p_ref[0] + p_ref[1]
    h3 = agg[:, :H] * inv_ref[...] + sf_ref[...]
    tab_ref[...] = jnp.dot(h3, wc_ref[...], preferred_element_type=_F32)


_combine_last = pl.pallas_call(
    _combine_last_body,
    out_shape=jax.ShapeDtypeStruct((NN, W128), _F32),
)

_BE = 8000


def _edge_mlp_body(gs, gd, ea, w1, b1, w2, b2, w3, b3, out):
    z1 = gs[:, :H] + gd[:, H:] + jnp.dot(ea[...], w1[...],
                                         preferred_element_type=_F32) + b1[...]
    z1 = jnp.maximum(z1, 0.0)
    z2 = jnp.maximum(jnp.dot(z1, w2[...],
                             preferred_element_type=_F32) + b2[...], 0.0)
    o = jnp.dot(z2, w3[...], preferred_element_type=_F32) + b3[...]
    out[...] = 1.0 / (1.0 + jnp.exp(-o))


_edge_mlp = pl.pallas_call(
    _edge_mlp_body,
    grid=(NE // _BE,),
    in_specs=[
        pl.BlockSpec((_BE, W128), lambda i: (i, 0)),
        pl.BlockSpec((_BE, W128), lambda i: (i, 0)),
        pl.BlockSpec((_BE, 16), lambda i: (i, 0)),
        pl.BlockSpec((16, H), lambda i: (0, 0)),
        pl.BlockSpec((1, H), lambda i: (0, 0)),
        pl.BlockSpec((H, 32), lambda i: (0, 0)),
        pl.BlockSpec((1, 32), lambda i: (0, 0)),
        pl.BlockSpec((32, 1), lambda i: (0, 0)),
        pl.BlockSpec((1, 1), lambda i: (0, 0)),
    ],
    out_specs=pl.BlockSpec((_BE, 1), lambda i: (i, 0)),
    out_shape=jax.ShapeDtypeStruct((NE, 1), _F32),
)


def kernel(x, edge_index, edge_attr, Wl0, Wr0, b0, Wl1, Wr1, b1,
           Wl2, Wr2, b2, PW1, Pb1, PW2, Pb2, PW3, Pb3):
    npad = KPW * NW * CHE - NE
    srcp = jnp.concatenate([edge_index[0], jnp.zeros((npad,), jnp.int32)])
    dstp = jnp.concatenate(
        [edge_index[1],
         NN + (jnp.arange(npad, dtype=jnp.int32) % CHE)])
    src3 = srcp.reshape(KPW, NW, CHE).transpose(1, 0, 2)
    dst3 = dstp.reshape(KPW, NW, CHE).transpose(1, 0, 2)
    z2d = jnp.zeros((NN, W128), _F32)

    def wpad(Wl):
        return jnp.pad(Wl, ((0, 0), (0, W128 - H)))

    e64 = jnp.zeros((1, W128), _F32).at[0, H].set(1.0)
    WcP = jnp.concatenate([PW1[:H], PW1[H:2 * H]], axis=1)

    tab0, sf0 = _mm0(x, wpad(Wl0), e64, Wr0, b0.reshape(1, H))
    p1 = _sc_agg(tab0, src3, dst3, z2d)
    tab1, sf1, inv = _combine_first(p1, sf0, wpad(Wl1), Wr1, b1.reshape(1, H))
    p2 = _sc_agg(tab1, src3, dst3, z2d)
    tab2, sf2 = _combine_mid(p2, inv, sf1, wpad(Wl2), Wr2, b2.reshape(1, H))
    p3 = _sc_agg(tab2, src3, dst3, z2d)
    huv = _combine_last(p3, inv, sf2, WcP)
    gs, gd = _sc_gather(huv, src3, dst3)
    return _edge_mlp(gs, gd, edge_attr, PW1[2 * H:], Pb1.reshape(1, H),
                     PW2, Pb2.reshape(1, 32), PW3, Pb3.reshape(1, 1))


# speedup vs baseline: 1.0330x; 1.0330x over previous
import jax
import jax.numpy as jnp
from jax import lax
from jax.experimental import pallas as pl
from jax.experimental.pallas import tpu as pltpu
from jax.experimental.pallas import tpu_sc as plsc

NN = 10000
NE = 320000
DIN = 128
H = 64
W128 = 128
NC = 2
NS = 16
NW = NC * NS
CHE = 128
NCHUNK = NE // CHE
_F32 = jnp.float32

_MESH = plsc.VectorSubcoreMesh(core_axis_name="c", subcore_axis_name="s")

_RPW = 624
_TAIL = NN - NS * _RPW


NFULL = NCHUNK // NW
NEXTRA = NCHUNK - NFULL * NW


def _worker_ids():
    c = lax.axis_index("c")
    s = lax.axis_index("s")
    return c, s, s * NC + c


def _zero_acc(z2d, acc, s):
    pltpu.sync_copy(z2d.at[pl.ds(s * _RPW, _RPW)],
                    acc.at[pl.ds(s * _RPW, _RPW)])

    @pl.when(s == 0)
    def _():
        pltpu.sync_copy(z2d.at[pl.ds(NS * _RPW, _TAIL)],
                        acc.at[pl.ds(NS * _RPW, _TAIL)])


KPW = 80
PH = 40


def _sc_agg_body(tab, srcp2, dstp2, z2d,
                 agg_out,
                 idx_sa, idx_da, idx_sb, idx_db, rows_a, rows_b,
                 sem_a, sem_b, sem_sa, sem_sb, acc):
    c, s, w = _worker_ids()
    _zero_acc(z2d, acc, s)
    plsc.subcore_barrier()

    def ld(idx_s, idx_d, k):
        ch = w + k * NW
        pltpu.sync_copy(srcp2.at[ch], idx_s)
        pltpu.sync_copy(dstp2.at[ch], idx_d)

    ld(idx_sa, idx_da, 0)
    pltpu.async_copy(tab.at[idx_sa], rows_a, sem_a)
    ld(idx_sb, idx_db, 1)
    pltpu.async_copy(tab.at[idx_sb], rows_b, sem_b)

    def body(m, carry):
        kA = 2 * m
        kB = kA + 1
        pltpu.make_async_copy(tab.at[idx_sa], rows_a, sem_a).wait()
        pltpu.async_copy(rows_a, acc.at[idx_da], sem_sa, add=True)
        pltpu.make_async_copy(tab.at[idx_sb], rows_b, sem_b).wait()
        pltpu.async_copy(rows_b, acc.at[idx_db], sem_sb, add=True)

        @pl.when(m < KPW // 2 - 1)
        def _():
            pltpu.make_async_copy(rows_a, acc.at[idx_da], sem_sa).wait()
            ld(idx_sa, idx_da, kA + 2)
            pltpu.async_copy(tab.at[idx_sa], rows_a, sem_a)
            pltpu.make_async_copy(rows_b, acc.at[idx_db], sem_sb).wait()
            ld(idx_sb, idx_db, kB + 2)
            pltpu.async_copy(tab.at[idx_sb], rows_b, sem_b)

        @pl.when(m == KPW // 2 - 1)
        def _():
            pltpu.make_async_copy(rows_a, acc.at[idx_da], sem_sa).wait()
            pltpu.make_async_copy(rows_b, acc.at[idx_db], sem_sb).wait()

        return carry

    lax.fori_loop(0, KPW // 2, body, 0)

    plsc.subcore_barrier()

    pltpu.sync_copy(acc.at[pl.ds(s * _RPW, _RPW)],
                    agg_out.at[c, pl.ds(s * _RPW, _RPW)])

    @pl.when(s == 0)
    def _():
        pltpu.sync_copy(acc.at[pl.ds(NS * _RPW, _TAIL)],
                        agg_out.at[c, pl.ds(NS * _RPW, _TAIL)])


def _sc_gather_body(tab, src3, dst3,
                    gs_out, gd_out,
                    idxs_all, idxd_all,
                    rows_sa, rows_sb, rows_da, rows_db,
                    sem_sa, sem_sb, sem_da, sem_db):
    c, s, w = _worker_ids()
    pltpu.sync_copy(src3.at[w], idxs_all)
    pltpu.sync_copy(dst3.at[w], idxd_all)

    pltpu.async_copy(tab.at[idxs_all.at[0]], rows_sa, sem_sa)
    pltpu.async_copy(tab.at[idxd_all.at[0]], rows_da, sem_da)
    pltpu.async_copy(tab.at[idxs_all.at[1]], rows_sb, sem_sb)
    pltpu.async_copy(tab.at[idxd_all.at[1]], rows_db, sem_db)

    def chunk_of(k):
        return k * NW + w

    def body(m, carry):
        kA = 2 * m
        kB = kA + 1
        baseA = chunk_of(kA) * CHE
        baseB = chunk_of(kB) * CHE
        pltpu.make_async_copy(tab.at[idxs_all.at[kA]], rows_sa, sem_sa).wait()
        pltpu.sync_copy(rows_sa, gs_out.at[pl.ds(baseA, CHE)])
        pltpu.make_async_copy(tab.at[idxd_all.at[kA]], rows_da, sem_da).wait()
        pltpu.sync_copy(rows_da, gd_out.at[pl.ds(baseA, CHE)])

        @pl.when(m < NFULL // 2 - 1)
        def _():
            pltpu.async_copy(tab.at[idxs_all.at[kA + 2]], rows_sa, sem_sa)
            pltpu.async_copy(tab.at[idxd_all.at[kA + 2]], rows_da, sem_da)

        pltpu.make_async_copy(tab.at[idxs_all.at[kB]], rows_sb, sem_sb).wait()
        pltpu.sync_copy(rows_sb, gs_out.at[pl.ds(baseB, CHE)])
        pltpu.make_async_copy(tab.at[idxd_all.at[kB]], rows_db, sem_db).wait()
        pltpu.sync_copy(rows_db, gd_out.at[pl.ds(baseB, CHE)])

        @pl.when(m < NFULL // 2 - 1)
        def _():
            pltpu.async_copy(tab.at[idxs_all.at[kB + 2]], rows_sb, sem_sb)
            pltpu.async_copy(tab.at[idxd_all.at[kB + 2]], rows_db, sem_db)

        return carry

    lax.fori_loop(0, NFULL // 2, body, 0)

    @pl.when(w < NEXTRA)
    def _():
        base = (NFULL * NW + w) * CHE
        pltpu.async_copy(tab.at[idxs_all.at[NFULL]], rows_sa, sem_sa).wait()
        pltpu.sync_copy(rows_sa, gs_out.at[pl.ds(base, CHE)])
        pltpu.async_copy(tab.at[idxd_all.at[NFULL]], rows_da, sem_da).wait()
        pltpu.sync_copy(rows_da, gd_out.at[pl.ds(base, CHE)])


_sc_agg = pl.kernel(
    _sc_agg_body,
    out_type=jax.ShapeDtypeStruct((NC, NN, W128), _F32),
    mesh=_MESH,
    scratch_types=[
        pltpu.VMEM((CHE,), jnp.int32),
        pltpu.VMEM((CHE,), jnp.int32),
        pltpu.VMEM((CHE,), jnp.int32),
        pltpu.VMEM((CHE,), jnp.int32),
        pltpu.VMEM((CHE, W128), _F32),
        pltpu.VMEM((CHE, W128), _F32),
        pltpu.SemaphoreType.DMA,
        pltpu.SemaphoreType.DMA,
        pltpu.SemaphoreType.DMA,
        pltpu.SemaphoreType.DMA,
        pltpu.VMEM_SHARED((NN + CHE, W128), _F32),
    ],
)

_sc_gather = pl.kernel(
    _sc_gather_body,
    out_type=(jax.ShapeDtypeStruct((NE, W128), _F32),
              jax.ShapeDtypeStruct((NE, W128), _F32)),
    mesh=_MESH,
    scratch_types=[
        pltpu.VMEM((KPW, CHE), jnp.int32),
        pltpu.VMEM((KPW, CHE), jnp.int32),
        pltpu.VMEM((CHE, W128), _F32),
        pltpu.VMEM((CHE, W128), _F32),
        pltpu.VMEM((CHE, W128), _F32),
        pltpu.VMEM((CHE, W128), _F32),
        pltpu.SemaphoreType.DMA,
        pltpu.SemaphoreType.DMA,
        pltpu.SemaphoreType.DMA,
        pltpu.SemaphoreType.DMA,
    ],
)


def _mm0_body(x_ref, wpad_ref, e64_ref, wr_ref, br_ref, tab_ref, sf_ref):
    x = x_ref[...]
    tab_ref[...] = jnp.dot(x, wpad_ref[...],
                           preferred_element_type=_F32) + e64_ref[...]
    sf_ref[...] = jnp.dot(x, wr_ref[...],
                          preferred_element_type=_F32) + br_ref[...]


_mm0 = pl.pallas_call(
    _mm0_body,
    out_shape=(jax.ShapeDtypeStruct((NN, W128), _F32),
               jax.ShapeDtypeStruct((NN, H), _F32)),
)


def _combine_first_body(p_ref, sf_ref, wpad_ref, wr_ref, br_ref,
                        tab_ref, sfo_ref, inv_ref):
    agg = p_ref[0] + p_ref[1]
    inv = 1.0 / jnp.maximum(agg[:, H:H + 1], 1.0)
    h = jnp.maximum(agg[:, :H] * inv + sf_ref[...], 0.0)
    tab_ref[...] = jnp.dot(h, wpad_ref[...], preferred_element_type=_F32)
    sfo_ref[...] = jnp.dot(h, wr_ref[...],
                           preferred_element_type=_F32) + br_ref[...]
    inv_ref[...] = inv


_combine_first = pl.pallas_call(
    _combine_first_body,
    out_shape=(jax.ShapeDtypeStruct((NN, W128), _F32),
               jax.ShapeDtypeStruct((NN, H), _F32),
               jax.ShapeDtypeStruct((NN, 1), _F32)),
)


def _combine_mid_body(p_ref, inv_ref, sf_ref, wpad_ref, wr_ref, br_ref,
                      tab_ref, sfo_ref):
    agg = p_ref[0] + p_ref[1]
    h = jnp.maximum(agg[:, :H] * inv_ref[...] + sf_ref[...], 0.0)
    tab_ref[...] = jnp.dot(h, wpad_ref[...], preferred_element_type=_F32)
    sfo_ref[...] = jnp.dot(h, wr_ref[...],
                           preferred_element_type=_F32) + br_ref[...]


_combine_mid = pl.pallas_call(
    _combine_mid_body,
    out_shape=(jax.ShapeDtypeStruct((NN, W128), _F32),
               jax.ShapeDtypeStruct((NN, H), _F32)),
)


def _combine_last_body(p_ref, inv_ref, sf_ref, wc_ref, tab_ref):
    agg = p_ref[0] + p_ref[1]
    h3 = agg[:, :H] * inv_ref[...] + sf_ref[...]
    tab_ref[...] = jnp.dot(h3, wc_ref[...], preferred_element_type=_F32)


_combine_last = pl.pallas_call(
    _combine_last_body,
    out_shape=jax.ShapeDtypeStruct((NN, W128), _F32),
)

_BE = 8000


def _edge_mlp_body(gs, gd, ea, w1, b1, w2, b2, w3, b3, out):
    z1 = gs[:, :H] + gd[:, H:] + jnp.dot(ea[...], w1[...],
                                         preferred_element_type=_F32) + b1[...]
    z1 = jnp.maximum(z1, 0.0)
    z2 = jnp.maximum(jnp.dot(z1, w2[...],
                             preferred_element_type=_F32) + b2[...], 0.0)
    o = jnp.dot(z2, w3[...], preferred_element_type=_F32) + b3[...]
    out[...] = 1.0 / (1.0 + jnp.exp(-o))


_edge_mlp = pl.pallas_call(
    _edge_mlp_body,
    grid=(NE // _BE,),
    in_specs=[
        pl.BlockSpec((_BE, W128), lambda i: (i, 0)),
        pl.BlockSpec((_BE, W128), lambda i: (i, 0)),
        pl.BlockSpec((_BE, 16), lambda i: (i, 0)),
        pl.BlockSpec((16, H), lambda i: (0, 0)),
        pl.BlockSpec((1, H), lambda i: (0, 0)),
        pl.BlockSpec((H, 32), lambda i: (0, 0)),
        pl.BlockSpec((1, 32), lambda i: (0, 0)),
        pl.BlockSpec((32, 1), lambda i: (0, 0)),
        pl.BlockSpec((1, 1), lambda i: (0, 0)),
    ],
    out_specs=pl.BlockSpec((_BE, 1), lambda i: (i, 0)),
    out_shape=jax.ShapeDtypeStruct((NE, 1), _F32),
)


def kernel(x, edge_index, edge_attr, Wl0, Wr0, b0, Wl1, Wr1, b1,
           Wl2, Wr2, b2, PW1, Pb1, PW2, Pb2, PW3, Pb3):
    npad = KPW * NW * CHE - NE
    srcp = jnp.concatenate([edge_index[0], jnp.zeros((npad,), jnp.int32)])
    dstp = jnp.concatenate(
        [edge_index[1],
         NN + (jnp.arange(npad, dtype=jnp.int32) % CHE)])
    srcp2 = srcp.reshape(KPW * NW, CHE)
    dstp2 = dstp.reshape(KPW * NW, CHE)
    src3 = srcp.reshape(KPW, NW, CHE).transpose(1, 0, 2)
    dst3 = dstp.reshape(KPW, NW, CHE).transpose(1, 0, 2)
    z2d = jnp.zeros((NN, W128), _F32)

    def wpad(Wl):
        return jnp.pad(Wl, ((0, 0), (0, W128 - H)))

    e64 = jnp.zeros((1, W128), _F32).at[0, H].set(1.0)
    WcP = jnp.concatenate([PW1[:H], PW1[H:2 * H]], axis=1)

    tab0, sf0 = _mm0(x, wpad(Wl0), e64, Wr0, b0.reshape(1, H))
    p1 = _sc_agg(tab0, srcp2, dstp2, z2d)
    tab1, sf1, inv = _combine_first(p1, sf0, wpad(Wl1), Wr1, b1.reshape(1, H))
    p2 = _sc_agg(tab1, srcp2, dstp2, z2d)
    tab2, sf2 = _combine_mid(p2, inv, sf1, wpad(Wl2), Wr2, b2.reshape(1, H))
    p3 = _sc_agg(tab2, srcp2, dstp2, z2d)
    huv = _combine_last(p3, inv, sf2, WcP)
    gs, gd = _sc_gather(huv, src3, dst3)
    return _edge_mlp(gs, gd, edge_attr, PW1[2 * H:], Pb1.reshape(1, H),
                     PW2, Pb2.reshape(1, 32), PW3, Pb3.reshape(1, 1))


# speedup vs baseline: 1.7977x; 1.7403x over previous
import jax
import jax.numpy as jnp
from jax import lax
from jax.experimental import pallas as pl
from jax.experimental.pallas import tpu as pltpu
from jax.experimental.pallas import tpu_sc as plsc

NN = 10000
NE = 320000
DIN = 128
H = 64
W128 = 128
NC = 2
NS = 16
NW = NC * NS
CHE = 128
NCHUNK = NE // CHE
_F32 = jnp.float32

_MESH = plsc.VectorSubcoreMesh(core_axis_name="c", subcore_axis_name="s")

_RPW = 624
_TAIL = NN - NS * _RPW


NFULL = NCHUNK // NW
NEXTRA = NCHUNK - NFULL * NW


def _worker_ids():
    c = lax.axis_index("c")
    s = lax.axis_index("s")
    return c, s, s * NC + c


def _zero_acc(z2d, acc, s):
    pltpu.sync_copy(z2d.at[pl.ds(s * _RPW, _RPW)],
                    acc.at[pl.ds(s * _RPW, _RPW)])

    @pl.when(s == 0)
    def _():
        pltpu.sync_copy(z2d.at[pl.ds(NS * _RPW, _TAIL)],
                        acc.at[pl.ds(NS * _RPW, _TAIL)])


KPW = 80
PH = 40


def _sc_agg_body(tab, srcp2, dstp2, z2d,
                 agg_out,
                 idx_sa, idx_da, idx_sb, idx_db, rows_a, rows_b,
                 sem_a, sem_b, sem_sa, sem_sb, acc):
    c, s, w = _worker_ids()
    _zero_acc(z2d, acc, s)
    plsc.subcore_barrier()

    def ld(idx_s, idx_d, k):
        ch = w + k * NW
        pltpu.sync_copy(srcp2.at[ch], idx_s)
        pltpu.sync_copy(dstp2.at[ch], idx_d)

    ld(idx_sa, idx_da, 0)
    pltpu.async_copy(tab.at[idx_sa], rows_a, sem_a)
    ld(idx_sb, idx_db, 1)
    pltpu.async_copy(tab.at[idx_sb], rows_b, sem_b)

    def body(m, carry):
        kA = 2 * m
        kB = kA + 1
        pltpu.make_async_copy(tab.at[idx_sa], rows_a, sem_a).wait()
        pltpu.async_copy(rows_a, acc.at[idx_da], sem_sa, add=True)
        pltpu.make_async_copy(tab.at[idx_sb], rows_b, sem_b).wait()
        pltpu.async_copy(rows_b, acc.at[idx_db], sem_sb, add=True)

        @pl.when(m < NFULL // 2 - 1)
        def _():
            pltpu.make_async_copy(rows_a, acc.at[idx_da], sem_sa).wait()
            ld(idx_sa, idx_da, kA + 2)
            pltpu.async_copy(tab.at[idx_sa], rows_a, sem_a)
            pltpu.make_async_copy(rows_b, acc.at[idx_db], sem_sb).wait()
            ld(idx_sb, idx_db, kB + 2)
            pltpu.async_copy(tab.at[idx_sb], rows_b, sem_b)

        @pl.when(m == NFULL // 2 - 1)
        def _():
            pltpu.make_async_copy(rows_a, acc.at[idx_da], sem_sa).wait()
            pltpu.make_async_copy(rows_b, acc.at[idx_db], sem_sb).wait()

        return carry

    lax.fori_loop(0, NFULL // 2, body, 0)

    @pl.when(w < NEXTRA)
    def _():
        ld(idx_sa, idx_da, NFULL)
        pltpu.async_copy(tab.at[idx_sa], rows_a, sem_a).wait()
        pltpu.sync_copy(rows_a, acc.at[idx_da], add=True)

    plsc.subcore_barrier()

    pltpu.sync_copy(acc.at[pl.ds(s * _RPW, _RPW)],
                    agg_out.at[c, pl.ds(s * _RPW, _RPW)])

    @pl.when(s == 0)
    def _():
        pltpu.sync_copy(acc.at[pl.ds(NS * _RPW, _TAIL)],
                        agg_out.at[c, pl.ds(NS * _RPW, _TAIL)])


def _sc_gather_body(tab, src3, dst3,
                    gs_out, gd_out,
                    idxs_all, idxd_all,
                    rows_sa, rows_sb, rows_da, rows_db,
                    sem_sa, sem_sb, sem_da, sem_db):
    c, s, w = _worker_ids()
    pltpu.sync_copy(src3.at[w], idxs_all)
    pltpu.sync_copy(dst3.at[w], idxd_all)

    pltpu.async_copy(tab.at[idxs_all.at[0]], rows_sa, sem_sa)
    pltpu.async_copy(tab.at[idxd_all.at[0]], rows_da, sem_da)
    pltpu.async_copy(tab.at[idxs_all.at[1]], rows_sb, sem_sb)
    pltpu.async_copy(tab.at[idxd_all.at[1]], rows_db, sem_db)

    def chunk_of(k):
        return k * NW + w

    def body(m, carry):
        kA = 2 * m
        kB = kA + 1
        baseA = chunk_of(kA) * CHE
        baseB = chunk_of(kB) * CHE
        pltpu.make_async_copy(tab.at[idxs_all.at[kA]], rows_sa, sem_sa).wait()
        pltpu.sync_copy(rows_sa, gs_out.at[pl.ds(baseA, CHE)])
        pltpu.make_async_copy(tab.at[idxd_all.at[kA]], rows_da, sem_da).wait()
        pltpu.sync_copy(rows_da, gd_out.at[pl.ds(baseA, CHE)])

        @pl.when(m < NFULL // 2 - 1)
        def _():
            pltpu.async_copy(tab.at[idxs_all.at[kA + 2]], rows_sa, sem_sa)
            pltpu.async_copy(tab.at[idxd_all.at[kA + 2]], rows_da, sem_da)

        pltpu.make_async_copy(tab.at[idxs_all.at[kB]], rows_sb, sem_sb).wait()
        pltpu.sync_copy(rows_sb, gs_out.at[pl.ds(baseB, CHE)])
        pltpu.make_async_copy(tab.at[idxd_all.at[kB]], rows_db, sem_db).wait()
        pltpu.sync_copy(rows_db, gd_out.at[pl.ds(baseB, CHE)])

        @pl.when(m < NFULL // 2 - 1)
        def _():
            pltpu.async_copy(tab.at[idxs_all.at[kB + 2]], rows_sb, sem_sb)
            pltpu.async_copy(tab.at[idxd_all.at[kB + 2]], rows_db, sem_db)

        return carry

    lax.fori_loop(0, NFULL // 2, body, 0)

    @pl.when(w < NEXTRA)
    def _():
        base = (NFULL * NW + w) * CHE
        pltpu.async_copy(tab.at[idxs_all.at[NFULL]], rows_sa, sem_sa).wait()
        pltpu.sync_copy(rows_sa, gs_out.at[pl.ds(base, CHE)])
        pltpu.async_copy(tab.at[idxd_all.at[NFULL]], rows_da, sem_da).wait()
        pltpu.sync_copy(rows_da, gd_out.at[pl.ds(base, CHE)])


_sc_agg = pl.kernel(
    _sc_agg_body,
    out_type=jax.ShapeDtypeStruct((NC, NN, W128), _F32),
    mesh=_MESH,
    scratch_types=[
        pltpu.VMEM((CHE,), jnp.int32),
        pltpu.VMEM((CHE,), jnp.int32),
        pltpu.VMEM((CHE,), jnp.int32),
        pltpu.VMEM((CHE,), jnp.int32),
        pltpu.VMEM((CHE, W128), _F32),
        pltpu.VMEM((CHE, W128), _F32),
        pltpu.SemaphoreType.DMA,
        pltpu.SemaphoreType.DMA,
        pltpu.SemaphoreType.DMA,
        pltpu.SemaphoreType.DMA,
        pltpu.VMEM_SHARED((NN + CHE, W128), _F32),
    ],
)

_sc_gather = pl.kernel(
    _sc_gather_body,
    out_type=(jax.ShapeDtypeStruct((NE, W128), _F32),
              jax.ShapeDtypeStruct((NE, W128), _F32)),
    mesh=_MESH,
    scratch_types=[
        pltpu.VMEM((KPW, CHE), jnp.int32),
        pltpu.VMEM((KPW, CHE), jnp.int32),
        pltpu.VMEM((CHE, W128), _F32),
        pltpu.VMEM((CHE, W128), _F32),
        pltpu.VMEM((CHE, W128), _F32),
        pltpu.VMEM((CHE, W128), _F32),
        pltpu.SemaphoreType.DMA,
        pltpu.SemaphoreType.DMA,
        pltpu.SemaphoreType.DMA,
        pltpu.SemaphoreType.DMA,
    ],
)


def _mm0_body(x_ref, wpad_ref, e64_ref, wr_ref, br_ref, tab_ref, sf_ref):
    x = x_ref[...]
    tab_ref[...] = jnp.dot(x, wpad_ref[...],
                           preferred_element_type=_F32) + e64_ref[...]
    sf_ref[...] = jnp.dot(x, wr_ref[...],
                          preferred_element_type=_F32) + br_ref[...]


_mm0 = pl.pallas_call(
    _mm0_body,
    out_shape=(jax.ShapeDtypeStruct((NN, W128), _F32),
               jax.ShapeDtypeStruct((NN, H), _F32)),
)


def _combine_first_body(p_ref, sf_ref, wpad_ref, wr_ref, br_ref,
                        tab_ref, sfo_ref, inv_ref):
    agg = p_ref[0] + p_ref[1]
    inv = 1.0 / jnp.maximum(agg[:, H:H + 1], 1.0)
    h = jnp.maximum(agg[:, :H] * inv + sf_ref[...], 0.0)
    tab_ref[...] = jnp.dot(h, wpad_ref[...], preferred_element_type=_F32)
    sfo_ref[...] = jnp.dot(h, wr_ref[...],
                           preferred_element_type=_F32) + br_ref[...]
    inv_ref[...] = inv


_combine_first = pl.pallas_call(
    _combine_first_body,
    out_shape=(jax.ShapeDtypeStruct((NN, W128), _F32),
               jax.ShapeDtypeStruct((NN, H), _F32),
               jax.ShapeDtypeStruct((NN, 1), _F32)),
)


def _combine_mid_body(p_ref, inv_ref, sf_ref, wpad_ref, wr_ref, br_ref,
                      tab_ref, sfo_ref):
    agg = p_ref[0] + p_ref[1]
    h = jnp.maximum(agg[:, :H] * inv_ref[...] + sf_ref[...], 0.0)
    tab_ref[...] = jnp.dot(h, wpad_ref[...], preferred_element_type=_F32)
    sfo_ref[...] = jnp.dot(h, wr_ref[...],
                           preferred_element_type=_F32) + br_ref[...]


_combine_mid = pl.pallas_call(
    _combine_mid_body,
    out_shape=(jax.ShapeDtypeStruct((NN, W128), _F32),
               jax.ShapeDtypeStruct((NN, H), _F32)),
)


def _combine_last_body(p_ref, inv_ref, sf_ref, wc_ref, tab_ref):
    agg = p_ref[0] + p_ref[1]
    h3 = agg[:, :H] * inv_ref[...] + sf_ref[...]
    tab_ref[...] = jnp.dot(h3, wc_ref[...], preferred_element_type=_F32)


_combine_last = pl.pallas_call(
    _combine_last_body,
    out_shape=jax.ShapeDtypeStruct((NN, W128), _F32),
)

_BE = 8000


def _edge_mlp_body(gs, gd, ea, w1, b1, w2, b2, w3, b3, out):
    z1 = gs[:, :H] + gd[:, H:] + jnp.dot(ea[...], w1[...],
                                         preferred_element_type=_F32) + b1[...]
    z1 = jnp.maximum(z1, 0.0)
    z2 = jnp.maximum(jnp.dot(z1, w2[...],
                             preferred_element_type=_F32) + b2[...], 0.0)
    o = jnp.dot(z2, w3[...], preferred_element_type=_F32) + b3[...]
    out[...] = 1.0 / (1.0 + jnp.exp(-o))


_edge_mlp = pl.pallas_call(
    _edge_mlp_body,
    grid=(NE // _BE,),
    in_specs=[
        pl.BlockSpec((_BE, W128), lambda i: (i, 0)),
        pl.BlockSpec((_BE, W128), lambda i: (i, 0)),
        pl.BlockSpec((_BE, 16), lambda i: (i, 0)),
        pl.BlockSpec((16, H), lambda i: (0, 0)),
        pl.BlockSpec((1, H), lambda i: (0, 0)),
        pl.BlockSpec((H, 32), lambda i: (0, 0)),
        pl.BlockSpec((1, 32), lambda i: (0, 0)),
        pl.BlockSpec((32, 1), lambda i: (0, 0)),
        pl.BlockSpec((1, 1), lambda i: (0, 0)),
    ],
    out_specs=pl.BlockSpec((_BE, 1), lambda i: (i, 0)),
    out_shape=jax.ShapeDtypeStruct((NE, 1), _F32),
)


def kernel(x, edge_index, edge_attr, Wl0, Wr0, b0, Wl1, Wr1, b1,
           Wl2, Wr2, b2, PW1, Pb1, PW2, Pb2, PW3, Pb3):
    npad = KPW * NW * CHE - NE
    srcp = jnp.concatenate([edge_index[0], jnp.zeros((npad,), jnp.int32)])
    dstp = jnp.concatenate(
        [edge_index[1],
         NN + (jnp.arange(npad, dtype=jnp.int32) % CHE)])
    srcp2 = srcp.reshape(KPW * NW, CHE)
    dstp2 = dstp.reshape(KPW * NW, CHE)
    src3 = srcp.reshape(KPW, NW, CHE).transpose(1, 0, 2)
    dst3 = dstp.reshape(KPW, NW, CHE).transpose(1, 0, 2)
    z2d = jnp.zeros((NN, W128), _F32)

    def wpad(Wl):
        return jnp.pad(Wl, ((0, 0), (0, W128 - H)))

    e64 = jnp.zeros((1, W128), _F32).at[0, H].set(1.0)
    WcP = jnp.concatenate([PW1[:H], PW1[H:2 * H]], axis=1)

    tab0, sf0 = _mm0(x, wpad(Wl0), e64, Wr0, b0.reshape(1, H))
    p1 = _sc_agg(tab0, srcp2, dstp2, z2d)
    tab1, sf1, inv = _combine_first(p1, sf0, wpad(Wl1), Wr1, b1.reshape(1, H))
    p2 = _sc_agg(tab1, srcp2, dstp2, z2d)
    tab2, sf2 = _combine_mid(p2, inv, sf1, wpad(Wl2), Wr2, b2.reshape(1, H))
    p3 = _sc_agg(tab2, srcp2, dstp2, z2d)
    huv = _combine_last(p3, inv, sf2, WcP)
    gs, gd = _sc_gather(huv, src3, dst3)
    return _edge_mlp(gs, gd, edge_attr, PW1[2 * H:], Pb1.reshape(1, H),
                     PW2, Pb2.reshape(1, 32), PW3, Pb3.reshape(1, 1))


# speedup vs baseline: 1.8895x; 1.0511x over previous
import jax
import jax.numpy as jnp
from jax import lax
from jax.experimental import pallas as pl
from jax.experimental.pallas import tpu as pltpu
from jax.experimental.pallas import tpu_sc as plsc

NN = 10000
NE = 320000
DIN = 128
D_EDGE = 16
H = 64
W128 = 128
NC = 2
NS = 16
NW = NC * NS
CHE = 128
NCHUNK = NE // CHE
_F32 = jnp.float32

_MESH = plsc.VectorSubcoreMesh(core_axis_name="c", subcore_axis_name="s")

_RPW = 624
_TAIL = NN - NS * _RPW


NFULL = NCHUNK // NW
NEXTRA = NCHUNK - NFULL * NW


def _worker_ids():
    c = lax.axis_index("c")
    s = lax.axis_index("s")
    return c, s, s * NC + c


def _zero_acc(z2d, acc, s):
    pltpu.sync_copy(z2d.at[pl.ds(s * _RPW, _RPW)],
                    acc.at[pl.ds(s * _RPW, _RPW)])

    @pl.when(s == 0)
    def _():
        pltpu.sync_copy(z2d.at[pl.ds(NS * _RPW, _TAIL)],
                        acc.at[pl.ds(NS * _RPW, _TAIL)])


KPW = 80
PH = 40


def _sc_agg_body(tab, srcp2, dstp2, z2d,
                 agg_out,
                 idx_sa, idx_da, idx_sb, idx_db, rows_a, rows_b,
                 sem_a, sem_b, sem_sa, sem_sb, acc):
    c, s, w = _worker_ids()
    _zero_acc(z2d, acc, s)
    plsc.subcore_barrier()

    def ld(idx_s, idx_d, k):
        ch = w + k * NW
        pltpu.sync_copy(srcp2.at[ch], idx_s)
        pltpu.sync_copy(dstp2.at[ch], idx_d)

    ld(idx_sa, idx_da, 0)
    pltpu.async_copy(tab.at[idx_sa], rows_a, sem_a)
    ld(idx_sb, idx_db, 1)
    pltpu.async_copy(tab.at[idx_sb], rows_b, sem_b)

    def body(m, carry):
        kA = 2 * m
        kB = kA + 1
        pltpu.make_async_copy(tab.at[idx_sa], rows_a, sem_a).wait()
        pltpu.async_copy(rows_a, acc.at[idx_da], sem_sa, add=True)
        pltpu.make_async_copy(tab.at[idx_sb], rows_b, sem_b).wait()
        pltpu.async_copy(rows_b, acc.at[idx_db], sem_sb, add=True)

        @pl.when(m < NFULL // 2 - 1)
        def _():
            pltpu.make_async_copy(rows_a, acc.at[idx_da], sem_sa).wait()
            ld(idx_sa, idx_da, kA + 2)
            pltpu.async_copy(tab.at[idx_sa], rows_a, sem_a)
            pltpu.make_async_copy(rows_b, acc.at[idx_db], sem_sb).wait()
            ld(idx_sb, idx_db, kB + 2)
            pltpu.async_copy(tab.at[idx_sb], rows_b, sem_b)

        @pl.when(m == NFULL // 2 - 1)
        def _():
            pltpu.make_async_copy(rows_a, acc.at[idx_da], sem_sa).wait()
            pltpu.make_async_copy(rows_b, acc.at[idx_db], sem_sb).wait()

        return carry

    lax.fori_loop(0, NFULL // 2, body, 0)

    @pl.when(w < NEXTRA)
    def _():
        ld(idx_sa, idx_da, NFULL)
        pltpu.async_copy(tab.at[idx_sa], rows_a, sem_a).wait()
        pltpu.sync_copy(rows_a, acc.at[idx_da], add=True)

    plsc.subcore_barrier()

    pltpu.sync_copy(acc.at[pl.ds(s * _RPW, _RPW)],
                    agg_out.at[c, pl.ds(s * _RPW, _RPW)])

    @pl.when(s == 0)
    def _():
        pltpu.sync_copy(acc.at[pl.ds(NS * _RPW, _TAIL)],
                        agg_out.at[c, pl.ds(NS * _RPW, _TAIL)])


NE2 = NE // 2
CH2 = CHE // 2


def _pack_add(rows_s, rows_d, sbuf):
    def body(e2, carry):
        for j in range(8):
            e = 2 * e2 + j // 4
            cc = (j % 4) * 16
            sbuf[e2, pl.ds(j * 16, 16)] = (
                rows_s[e, pl.ds(cc, 16)] + rows_d[e, pl.ds(H + cc, 16)])
        return carry

    lax.fori_loop(0, CH2, body, 0)


def _sc_gather_body(tab, srcp2, dstp2,
                    s2_out,
                    idx_sa, idx_da, idx_sb, idx_db,
                    rows_sa, rows_da, rows_sb, rows_db, sbuf_a, sbuf_b,
                    sem_sa, sem_da, sem_sb, sem_db, sem_wa, sem_wb):
    c, s, w = _worker_ids()

    def ld(idx_s, idx_d, k):
        ch = w + k * NW
        pltpu.sync_copy(srcp2.at[ch], idx_s)
        pltpu.sync_copy(dstp2.at[ch], idx_d)

    def out_slice(k):
        return s2_out.at[pl.ds((w + k * NW) * CH2, CH2)]

    ld(idx_sa, idx_da, 0)
    pltpu.async_copy(tab.at[idx_sa], rows_sa, sem_sa)
    pltpu.async_copy(tab.at[idx_da], rows_da, sem_da)
    ld(idx_sb, idx_db, 1)
    pltpu.async_copy(tab.at[idx_sb], rows_sb, sem_sb)
    pltpu.async_copy(tab.at[idx_db], rows_db, sem_db)

    def body(m, carry):
        kA = 2 * m
        kB = kA + 1
        pltpu.make_async_copy(tab.at[idx_sa], rows_sa, sem_sa).wait()
        pltpu.make_async_copy(tab.at[idx_da], rows_da, sem_da).wait()

        @pl.when(m > 0)
        def _():
            pltpu.make_async_copy(sbuf_a, out_slice(kA - 2), sem_wa).wait()

        _pack_add(rows_sa, rows_da, sbuf_a)
        pltpu.async_copy(sbuf_a, out_slice(kA), sem_wa)

        @pl.when(m < NFULL // 2 - 1)
        def _():
            ld(idx_sa, idx_da, kA + 2)
            pltpu.async_copy(tab.at[idx_sa], rows_sa, sem_sa)
            pltpu.async_copy(tab.at[idx_da], rows_da, sem_da)

        pltpu.make_async_copy(tab.at[idx_sb], rows_sb, sem_sb).wait()
        pltpu.make_async_copy(tab.at[idx_db], rows_db, sem_db).wait()

        @pl.when(m > 0)
        def _():
            pltpu.make_async_copy(sbuf_b, out_slice(kB - 2), sem_wb).wait()

        _pack_add(rows_sb, rows_db, sbuf_b)
        pltpu.async_copy(sbuf_b, out_slice(kB), sem_wb)

        @pl.when(m < NFULL // 2 - 1)
        def _():
            ld(idx_sb, idx_db, kB + 2)
            pltpu.async_copy(tab.at[idx_sb], rows_sb, sem_sb)
            pltpu.async_copy(tab.at[idx_db], rows_db, sem_db)

        return carry

    lax.fori_loop(0, NFULL // 2, body, 0)
    pltpu.make_async_copy(sbuf_a, out_slice(NFULL - 2), sem_wa).wait()
    pltpu.make_async_copy(sbuf_b, out_slice(NFULL - 1), sem_wb).wait()

    @pl.when(w < NEXTRA)
    def _():
        ld(idx_sa, idx_da, NFULL)
        pltpu.async_copy(tab.at[idx_sa], rows_sa, sem_sa).wait()
        pltpu.async_copy(tab.at[idx_da], rows_da, sem_da).wait()
        _pack_add(rows_sa, rows_da, sbuf_a)
        pltpu.sync_copy(sbuf_a, out_slice(NFULL))


_sc_agg = pl.kernel(
    _sc_agg_body,
    out_type=jax.ShapeDtypeStruct((NC, NN, W128), _F32),
    mesh=_MESH,
    scratch_types=[
        pltpu.VMEM((CHE,), jnp.int32),
        pltpu.VMEM((CHE,), jnp.int32),
        pltpu.VMEM((CHE,), jnp.int32),
        pltpu.VMEM((CHE,), jnp.int32),
        pltpu.VMEM((CHE, W128), _F32),
        pltpu.VMEM((CHE, W128), _F32),
        pltpu.SemaphoreType.DMA,
        pltpu.SemaphoreType.DMA,
        pltpu.SemaphoreType.DMA,
        pltpu.SemaphoreType.DMA,
        pltpu.VMEM_SHARED((NN, W128), _F32),
    ],
)

_sc_gather = pl.kernel(
    _sc_gather_body,
    out_type=jax.ShapeDtypeStruct((NE2, W128), _F32),
    mesh=_MESH,
    scratch_types=[
        pltpu.VMEM((CHE,), jnp.int32),
        pltpu.VMEM((CHE,), jnp.int32),
        pltpu.VMEM((CHE,), jnp.int32),
        pltpu.VMEM((CHE,), jnp.int32),
        pltpu.VMEM((CHE, W128), _F32),
        pltpu.VMEM((CHE, W128), _F32),
        pltpu.VMEM((CHE, W128), _F32),
        pltpu.VMEM((CHE, W128), _F32),
        pltpu.VMEM((CH2, W128), _F32),
        pltpu.VMEM((CH2, W128), _F32),
        pltpu.SemaphoreType.DMA,
        pltpu.SemaphoreType.DMA,
        pltpu.SemaphoreType.DMA,
        pltpu.SemaphoreType.DMA,
        pltpu.SemaphoreType.DMA,
        pltpu.SemaphoreType.DMA,
    ],
)


def _mm0_body(x_ref, wpad_ref, e64_ref, wr_ref, br_ref, tab_ref, sf_ref):
    x = x_ref[...]
    tab_ref[...] = jnp.dot(x, wpad_ref[...],
                           preferred_element_type=_F32) + e64_ref[...]
    sf_ref[...] = jnp.dot(x, wr_ref[...],
                          preferred_element_type=_F32) + br_ref[...]


_mm0 = pl.pallas_call(
    _mm0_body,
    out_shape=(jax.ShapeDtypeStruct((NN, W128), _F32),
               jax.ShapeDtypeStruct((NN, H), _F32)),
)


def _combine_first_body(p_ref, sf_ref, wpad_ref, wr_ref, br_ref,
                        tab_ref, sfo_ref, inv_ref):
    agg = p_ref[0] + p_ref[1]
    inv = 1.0 / jnp.maximum(agg[:, H:H + 1], 1.0)
    h = jnp.maximum(agg[:, :H] * inv + sf_ref[...], 0.0)
    tab_ref[...] = jnp.dot(h, wpad_ref[...], preferred_element_type=_F32)
    sfo_ref[...] = jnp.dot(h, wr_ref[...],
                           preferred_element_type=_F32) + br_ref[...]
    inv_ref[...] = inv


_combine_first = pl.pallas_call(
    _combine_first_body,
    out_shape=(jax.ShapeDtypeStruct((NN, W128), _F32),
               jax.ShapeDtypeStruct((NN, H), _F32),
               jax.ShapeDtypeStruct((NN, 1), _F32)),
)


def _combine_mid_body(p_ref, inv_ref, sf_ref, wpad_ref, wr_ref, br_ref,
                      tab_ref, sfo_ref):
    agg = p_ref[0] + p_ref[1]
    h = jnp.maximum(agg[:, :H] * inv_ref[...] + sf_ref[...], 0.0)
    tab_ref[...] = jnp.dot(h, wpad_ref[...], preferred_element_type=_F32)
    sfo_ref[...] = jnp.dot(h, wr_ref[...],
                           preferred_element_type=_F32) + br_ref[...]


_combine_mid = pl.pallas_call(
    _combine_mid_body,
    out_shape=(jax.ShapeDtypeStruct((NN, W128), _F32),
               jax.ShapeDtypeStruct((NN, H), _F32)),
)


def _combine_last_body(p_ref, inv_ref, sf_ref, wc_ref, tab_ref):
    agg = p_ref[0] + p_ref[1]
    h3 = agg[:, :H] * inv_ref[...] + sf_ref[...]
    tab_ref[...] = jnp.dot(h3, wc_ref[...], preferred_element_type=_F32)


_combine_last = pl.pallas_call(
    _combine_last_body,
    out_shape=jax.ShapeDtypeStruct((NN, W128), _F32),
)

_BE2 = 4000


def _edge_mlp_body(s2, ea2, w1, b1, w2, b2, w3, b3, out):
    z1 = s2[...] + jnp.dot(ea2[...], w1[...],
                           preferred_element_type=_F32) + b1[...]
    z1 = jnp.maximum(z1, 0.0)
    z2 = jnp.maximum(jnp.dot(z1, w2[...],
                             preferred_element_type=_F32) + b2[...], 0.0)
    o = jnp.dot(z2, w3[...], preferred_element_type=_F32) + b3[...]
    out[...] = 1.0 / (1.0 + jnp.exp(-o))


_edge_mlp = pl.pallas_call(
    _edge_mlp_body,
    grid=(NE2 // _BE2,),
    in_specs=[
        pl.BlockSpec((_BE2, W128), lambda i: (i, 0)),
        pl.BlockSpec((_BE2, 32), lambda i: (i, 0)),
        pl.BlockSpec((32, W128), lambda i: (0, 0)),
        pl.BlockSpec((1, W128), lambda i: (0, 0)),
        pl.BlockSpec((W128, 2 * 32), lambda i: (0, 0)),
        pl.BlockSpec((1, 2 * 32), lambda i: (0, 0)),
        pl.BlockSpec((2 * 32, 2), lambda i: (0, 0)),
        pl.BlockSpec((1, 2), lambda i: (0, 0)),
    ],
    out_specs=pl.BlockSpec((_BE2, 2), lambda i: (i, 0)),
    out_shape=jax.ShapeDtypeStruct((NE2, 2), _F32),
)


def kernel(x, edge_index, edge_attr, Wl0, Wr0, b0, Wl1, Wr1, b1,
           Wl2, Wr2, b2, PW1, Pb1, PW2, Pb2, PW3, Pb3):
    npad = KPW * NW * CHE - NE
    srcp = jnp.concatenate([edge_index[0], jnp.zeros((npad,), jnp.int32)])
    dstp = jnp.concatenate([edge_index[1], jnp.zeros((npad,), jnp.int32)])
    srcp2 = srcp.reshape(KPW * NW, CHE)
    dstp2 = dstp.reshape(KPW * NW, CHE)
    z2d = jnp.zeros((NN, W128), _F32)

    def wpad(Wl):
        return jnp.pad(Wl, ((0, 0), (0, W128 - H)))

    e64 = jnp.zeros((1, W128), _F32).at[0, H].set(1.0)
    WcP = jnp.concatenate([PW1[:H], PW1[H:2 * H]], axis=1)

    tab0, sf0 = _mm0(x, wpad(Wl0), e64, Wr0, b0.reshape(1, H))
    p1 = _sc_agg(tab0, srcp2, dstp2, z2d)
    tab1, sf1, inv = _combine_first(p1, sf0, wpad(Wl1), Wr1, b1.reshape(1, H))
    p2 = _sc_agg(tab1, srcp2, dstp2, z2d)
    tab2, sf2 = _combine_mid(p2, inv, sf1, wpad(Wl2), Wr2, b2.reshape(1, H))
    p3 = _sc_agg(tab2, srcp2, dstp2, z2d)
    huv = _combine_last(p3, inv, sf2, WcP)
    s2 = _sc_gather(huv, srcp2, dstp2)

    def blockdiag(wmat):
        a, b = wmat.shape
        z = jnp.zeros((a, b), _F32)
        return jnp.concatenate(
            [jnp.concatenate([wmat, z], axis=1),
             jnp.concatenate([z, wmat], axis=1)], axis=0)

    ea2 = edge_attr.reshape(NE2, 2 * D_EDGE)
    w1d = blockdiag(PW1[2 * H:])
    b1d = jnp.concatenate([Pb1, Pb1]).reshape(1, W128)
    w2d = blockdiag(PW2)
    b2d = jnp.concatenate([Pb2, Pb2]).reshape(1, 64)
    w3d = blockdiag(PW3)
    b3d = jnp.concatenate([Pb3, Pb3]).reshape(1, 2)
    out2 = _edge_mlp(s2, ea2, w1d, b1d, w2d, b2d, w3d, b3d)
    return out2.reshape(NE, 1)
